# baseline (speedup 1.0000x reference)
import functools

import jax
import jax.numpy as jnp
from jax import lax
from jax.experimental import pallas as pl
from jax.experimental.pallas import tpu as pltpu
from jax.experimental.pallas import tpu_sc as plsc

N_IMG = 64
N_POOL = 50
IMG_ELEMS = 256 * 256 * 3

NC, NS = 2, 16
NW = NC * NS
SLICE_ELEMS = IMG_ELEMS // NW
NBUF = 4


def kernel(inputs, pool, coins, swap_idx):
    inputs_r = inputs.reshape(N_IMG, NW, SLICE_ELEMS)
    pool_r = pool.reshape(N_POOL, NW, SLICE_ELEMS)
    mesh = plsc.VectorSubcoreMesh(core_axis_name="c", subcore_axis_name="s")

    scratch = [pltpu.VMEM((N_IMG,), jnp.int32),
               pltpu.VMEM((N_IMG,), jnp.int32),
               pltpu.VMEM((NBUF, SLICE_ELEMS), jnp.float32)]
    scratch += [pltpu.SemaphoreType.DMA] * (2 * NBUF)

    @functools.partial(
        pl.kernel, mesh=mesh,
        out_type=jax.ShapeDtypeStruct((N_IMG, NW, SLICE_ELEMS), jnp.float32),
        scratch_types=scratch,
    )
    def body(inputs_hbm, pool_hbm, coins_hbm, swap_hbm, out_hbm,
             coins_v, swap_v, bufs, *sems):
        sems_in, sems_out = sems[:NBUF], sems[NBUF:]
        sl = lax.axis_index("s") * NC + lax.axis_index("c")

        pltpu.sync_copy(coins_hbm, coins_v)
        pltpu.sync_copy(swap_hbm, swap_v)
        lane = lax.broadcasted_iota(jnp.int32, (16,), 0)
        coins_vecs = [coins_v[pl.ds(16 * k, 16)] for k in range(4)]
        swap_vecs = [swap_v[pl.ds(16 * k, 16)] for k in range(4)]
        idx_vecs = [lane + jnp.int32(16 * k) for k in range(4)]
        perms = [jnp.bitwise_xor(lane, sh) for sh in (8, 4, 2, 1)]
        gdims = lax.GatherDimensionNumbers(
            offset_dims=(), collapsed_slice_dims=(0,), start_index_map=(0,))

        def lane_gather(v, p):
            return lax.gather(v, p[:, None], gdims, slice_sizes=(1,),
                              mode=lax.GatherScatterMode.PROMISE_IN_BOUNDS)

        def routing(i):
            c_i = coins_vecs[i // 16][i % 16]
            j_i = swap_vecs[i // 16][i % 16]
            if i == 0:
                use_pool = c_i == 0
                src = jnp.where(use_pool, j_i, jnp.int32(0))
                return use_pool, src
            acc = jnp.full((16,), -1, jnp.int32)
            for k in range(min(4, (i + 15) // 16)):
                m = ((swap_vecs[k] == j_i)
                     & (coins_vecs[k] == 0)
                     & (idx_vecs[k] < i))
                acc = jnp.maximum(acc, jnp.where(m, idx_vecs[k], -1))
            for p in perms:
                acc = jnp.maximum(acc, lane_gather(acc, p))
            prev = acc[0]
            use_pool = (c_i == 0) & (prev < 0)
            src = jnp.where(c_i == 0,
                            jnp.where(prev < 0, j_i, prev),
                            jnp.int32(i))
            return use_pool, src

        def start_in(i, slot):
            use_pool, src = routing(i)

            @pl.when(use_pool)
            def _():
                pltpu.async_copy(pool_hbm.at[src, sl], bufs.at[slot],
                                 sems_in[slot])

            @pl.when(jnp.logical_not(use_pool))
            def _():
                pltpu.async_copy(inputs_hbm.at[src, sl], bufs.at[slot],
                                 sems_in[slot])

        def wait_in(slot):
            pltpu.make_async_copy(inputs_hbm.at[0, sl], bufs.at[slot],
                                  sems_in[slot]).wait()

        def wait_out(slot):
            pltpu.make_async_copy(bufs.at[slot], out_hbm.at[0, sl],
                                  sems_out[slot]).wait()

        pre = min(NBUF - 1, N_IMG)
        for li in range(pre):
            start_in(li, li % NBUF)

        for li in range(N_IMG):
            slot = li % NBUF
            wait_in(slot)
            pltpu.async_copy(bufs.at[slot], out_hbm.at[li, sl],
                             sems_out[slot])
            nli = li + pre
            if nli < N_IMG:
                nslot = nli % NBUF
                if nli >= NBUF:
                    wait_out(nslot)
                start_in(nli, nslot)

        for slot in range(min(NBUF, N_IMG)):
            wait_out(slot)

    out = body(inputs_r, pool_r, coins, swap_idx)
    return out.reshape(N_IMG, 256, 256, 3)
